# baseline (speedup 1.0000x reference)
import functools

import jax
import jax.numpy as jnp
from jax import lax
from jax.experimental import pallas as pl
from jax.experimental.pallas import tpu as pltpu
from jax.experimental.pallas import tpu_sc as plsc

_N = 10000
_E = 320000
_D = 128
_DH = 64

_NC = 2
_NS = 16
_NW = _NC * _NS
_B = 128
_CHT = 160
_CHA = 152
_CHB = _CHT - _CHA
_CHD = _CHT // 2
_NCH = _NS * _CHT
_EP = _NCH * _B
_NP = 10112
_RPT = _NP // _NS
_DW = 16
_TB = 632
_ZR = 79


def _deg_body(src_hbm, dst_hbm, ones_hbm, zdeg_hbm, dego_hbm, degi_hbm,
              idx_s, idx_d, ones_v, dego_sh, degi_sh):
    cid = lax.axis_index("c")
    sid = lax.axis_index("s")
    wid = cid * _NS + sid
    base = wid * _CHD
    pltpu.sync_copy(src_hbm.at[pl.ds(base, _CHD)], idx_s)
    pltpu.sync_copy(dst_hbm.at[pl.ds(base, _CHD)], idx_d)
    pltpu.sync_copy(ones_hbm, ones_v)
    r0 = sid * _RPT
    pltpu.sync_copy(zdeg_hbm, dego_sh.at[pl.ds(r0, _RPT)])
    pltpu.sync_copy(zdeg_hbm, degi_sh.at[pl.ds(r0, _RPT)])
    plsc.subcore_barrier()

    def step(c, carry):
        pltpu.sync_copy(ones_v, dego_sh.at[idx_s.at[c]], add=True)
        pltpu.sync_copy(ones_v, degi_sh.at[idx_d.at[c]], add=True)
        return carry

    lax.fori_loop(0, _CHD, step, 0)
    plsc.subcore_barrier()
    pltpu.sync_copy(dego_sh.at[pl.ds(r0, _RPT)],
                    dego_hbm.at[cid].at[pl.ds(r0, _RPT)])
    pltpu.sync_copy(degi_sh.at[pl.ds(r0, _RPT)],
                    degi_hbm.at[cid].at[pl.ds(r0, _RPT)])


@functools.cache
def _deg_call():
    return pl.kernel(
        _deg_body,
        out_type=(
            jax.ShapeDtypeStruct((_NC, _NP, _DW), jnp.float32),
            jax.ShapeDtypeStruct((_NC, _NP, _DW), jnp.float32),
        ),
        mesh=plsc.VectorSubcoreMesh(core_axis_name="c", subcore_axis_name="s"),
        compiler_params=pltpu.CompilerParams(use_tc_tiling_on_sc=False),
        scratch_types=[
            pltpu.VMEM((_CHD, _B), jnp.int32),
            pltpu.VMEM((_CHD, _B), jnp.int32),
            pltpu.VMEM((_B, _DW), jnp.float32),
            pltpu.VMEM_SHARED((_NP, _DW), jnp.float32),
            pltpu.VMEM_SHARED((_NP, _DW), jnp.float32),
        ],
    )


def _agg_body(h_hbm, src_hbm, dst_hbm, agg_hbm,
              idx_s, idx_d, rows, zbuf, agg_sh, sem0, sem1):
    cid = lax.axis_index("c")
    sid = lax.axis_index("s")
    r0 = sid * _RPT
    sems = (sem0, sem1)

    z16 = jnp.zeros((16,), jnp.float32)

    def zstep(k, carry):
        zbuf[k // 4, pl.ds((k % 4) * 16, 16)] = z16
        return carry

    lax.fori_loop(0, _ZR * 4, zstep, 0)

    def run(nch, base):
        pltpu.sync_copy(src_hbm.at[pl.ds(base, nch)], idx_s.at[pl.ds(0, nch)])
        pltpu.sync_copy(dst_hbm.at[pl.ds(base, nch)], idx_d.at[pl.ds(0, nch)])
        for half in range(2):
            hh = h_hbm.at[half]
            for k in range(_RPT // _ZR):
                pltpu.sync_copy(zbuf, agg_sh.at[pl.ds(r0 + k * _ZR, _ZR)])
            plsc.subcore_barrier()

            pltpu.async_copy(hh.at[idx_s.at[0]], rows.at[0], sems[0])
            pltpu.async_copy(hh.at[idx_s.at[1]], rows.at[1], sems[1])

            def step(i, carry):
                c0 = 2 * i
                for b in range(2):
                    c = c0 + b
                    pltpu.make_async_copy(hh.at[idx_s.at[c]], rows.at[b],
                                          sems[b]).wait()
                    pltpu.sync_copy(rows.at[b], agg_sh.at[idx_d.at[c]],
                                    add=True)
                    pltpu.async_copy(hh.at[idx_s.at[c + 2]], rows.at[b],
                                     sems[b])
                return carry

            lax.fori_loop(0, nch // 2 - 1, step, 0)
            for b in range(2):
                c = nch - 2 + b
                pltpu.make_async_copy(hh.at[idx_s.at[c]], rows.at[b],
                                      sems[b]).wait()
                pltpu.sync_copy(rows.at[b], agg_sh.at[idx_d.at[c]], add=True)

            plsc.subcore_barrier()
            pltpu.sync_copy(agg_sh.at[pl.ds(r0, _RPT)],
                            agg_hbm.at[half].at[cid].at[pl.ds(r0, _RPT)])
            plsc.subcore_barrier()

    lax.cond(cid == 0,
             lambda: run(_CHA, sid * _CHA),
             lambda: run(_CHB, _NS * _CHA + sid * _CHB))


@functools.cache
def _agg_call():
    chm = max(_CHA, _CHB)
    return pl.kernel(
        _agg_body,
        out_type=jax.ShapeDtypeStruct((2, _NC, _NP, _DH), jnp.float32),
        mesh=plsc.VectorSubcoreMesh(core_axis_name="c", subcore_axis_name="s"),
        compiler_params=pltpu.CompilerParams(use_tc_tiling_on_sc=False),
        scratch_types=[
            pltpu.VMEM((chm, _B), jnp.int32),
            pltpu.VMEM((chm, _B), jnp.int32),
            pltpu.VMEM((2, _B, _DH), jnp.float32),
            pltpu.VMEM((_ZR, _DH), jnp.float32),
            pltpu.VMEM_SHARED((_NP, _DH), jnp.float32),
            pltpu.SemaphoreType.DMA,
            pltpu.SemaphoreType.DMA,
        ],
    )


def _mat_body(x_ref, w_ref, dego_ref, o_ref):
    d = jnp.sum(dego_ref[...], axis=(0, 2))
    s = lax.rsqrt(jnp.maximum(d, 1.0))
    h = jnp.dot(x_ref[...], w_ref[...], preferred_element_type=jnp.float32)
    h = h * s[:, None]
    o_ref[0] = h[:, :_DH]
    o_ref[1] = h[:, _DH:]


def _mat_call(xp, w, dego):
    return pl.pallas_call(
        _mat_body,
        grid=(_NP // _TB,),
        in_specs=[
            pl.BlockSpec((_TB, _D), lambda i: (i, 0)),
            pl.BlockSpec((_D, _D), lambda i: (0, 0)),
            pl.BlockSpec((_NC, _TB, _DW), lambda i: (0, i, 0)),
        ],
        out_specs=pl.BlockSpec((2, _TB, _DH), lambda i: (0, i, 0)),
        out_shape=jax.ShapeDtypeStruct((2, _NP, _DH), jnp.float32),
    )(xp, w, dego)


def _fin_body(agg_ref, degi_ref, b_ref, o_ref):
    a = jnp.concatenate(
        [agg_ref[0, 0] + agg_ref[0, 1], agg_ref[1, 0] + agg_ref[1, 1]],
        axis=-1)
    d = jnp.sum(degi_ref[...], axis=(0, 2))
    s = lax.rsqrt(jnp.maximum(d, 1.0))
    o_ref[...] = jnp.maximum(a * s[:, None] + b_ref[...], 0.0)


def _fin_call(aggp, degi, b2):
    return pl.pallas_call(
        _fin_body,
        grid=(_NP // _TB,),
        in_specs=[
            pl.BlockSpec((2, _NC, _TB, _DH), lambda i: (0, 0, i, 0)),
            pl.BlockSpec((_NC, _TB, _DW), lambda i: (0, i, 0)),
            pl.BlockSpec((1, _D), lambda i: (0, 0)),
        ],
        out_specs=pl.BlockSpec((_TB, _D), lambda i: (i, 0)),
        out_shape=jax.ShapeDtypeStruct((_NP, _D), jnp.float32),
    )(aggp, degi, b2)


def kernel(x, edge_index, W, b):
    pad = _EP - _E
    srcp = jnp.concatenate(
        [edge_index[0], jnp.full((pad,), _N, jnp.int32)]).reshape(_NCH, _B)
    dstp = jnp.concatenate(
        [edge_index[1], jnp.full((pad,), _N, jnp.int32)]).reshape(_NCH, _B)
    xp = jnp.zeros((_NP, _D), jnp.float32).at[:_N].set(x)
    ones_blk = jnp.zeros((_B, _DW), jnp.float32).at[:, 0].set(1.0)
    zdeg = jnp.zeros((_RPT, _DW), jnp.float32)

    dego, degi = _deg_call()(srcp, dstp, ones_blk, zdeg)
    hp = _mat_call(xp, W, dego)
    aggp = _agg_call()(hp, srcp, dstp)
    outp = _fin_call(aggp, degi, b.reshape(1, _D))
    return outp[:_N]


# speedup vs baseline: 1.0216x; 1.0216x over previous
import functools

import jax
import jax.numpy as jnp
from jax import lax
from jax.experimental import pallas as pl
from jax.experimental.pallas import tpu as pltpu
from jax.experimental.pallas import tpu_sc as plsc

_N = 10000
_E = 320000
_D = 128
_DH = 64

_NC = 2
_NS = 16
_NW = _NC * _NS
_B = 128
_CHT = 160
_CHA = 148
_CHB = _CHT - _CHA
_CHD = _CHT // 2
_NCH = _NS * _CHT
_EP = _NCH * _B
_NP = 10112
_RPT = _NP // _NS
_DW = 16
_TB = 632
_ZR = 79


def _deg_body(src_hbm, dst_hbm, ones_hbm, zdeg_hbm, dego_hbm, degi_hbm,
              idx_s, idx_d, ones_v, dego_sh, degi_sh):
    cid = lax.axis_index("c")
    sid = lax.axis_index("s")
    wid = cid * _NS + sid
    base = wid * _CHD
    pltpu.sync_copy(src_hbm.at[pl.ds(base, _CHD)], idx_s)
    pltpu.sync_copy(dst_hbm.at[pl.ds(base, _CHD)], idx_d)
    pltpu.sync_copy(ones_hbm, ones_v)
    r0 = sid * _RPT
    pltpu.sync_copy(zdeg_hbm, dego_sh.at[pl.ds(r0, _RPT)])
    pltpu.sync_copy(zdeg_hbm, degi_sh.at[pl.ds(r0, _RPT)])
    plsc.subcore_barrier()

    def step(c, carry):
        pltpu.sync_copy(ones_v, dego_sh.at[idx_s.at[c]], add=True)
        pltpu.sync_copy(ones_v, degi_sh.at[idx_d.at[c]], add=True)
        return carry

    lax.fori_loop(0, _CHD, step, 0)
    plsc.subcore_barrier()
    pltpu.sync_copy(dego_sh.at[pl.ds(r0, _RPT)],
                    dego_hbm.at[cid].at[pl.ds(r0, _RPT)])
    pltpu.sync_copy(degi_sh.at[pl.ds(r0, _RPT)],
                    degi_hbm.at[cid].at[pl.ds(r0, _RPT)])


@functools.cache
def _deg_call():
    return pl.kernel(
        _deg_body,
        out_type=(
            jax.ShapeDtypeStruct((_NC, _NP, _DW), jnp.float32),
            jax.ShapeDtypeStruct((_NC, _NP, _DW), jnp.float32),
        ),
        mesh=plsc.VectorSubcoreMesh(core_axis_name="c", subcore_axis_name="s"),
        compiler_params=pltpu.CompilerParams(use_tc_tiling_on_sc=False),
        scratch_types=[
            pltpu.VMEM((_CHD, _B), jnp.int32),
            pltpu.VMEM((_CHD, _B), jnp.int32),
            pltpu.VMEM((_B, _DW), jnp.float32),
            pltpu.VMEM_SHARED((_NP, _DW), jnp.float32),
            pltpu.VMEM_SHARED((_NP, _DW), jnp.float32),
        ],
    )


def _agg_body(h_hbm, src_hbm, dst_hbm, agg_hbm,
              idx_s, idx_d, rows, zbuf, agg_sh, sem0, sem1):
    cid = lax.axis_index("c")
    sid = lax.axis_index("s")
    r0 = sid * _RPT
    sems = (sem0, sem1)

    z16 = jnp.zeros((16,), jnp.float32)

    def zstep(k, carry):
        zbuf[k // 4, pl.ds((k % 4) * 16, 16)] = z16
        return carry

    lax.fori_loop(0, _ZR * 4, zstep, 0)

    def run(nch, base):
        pltpu.sync_copy(src_hbm.at[pl.ds(base, nch)], idx_s.at[pl.ds(0, nch)])
        pltpu.sync_copy(dst_hbm.at[pl.ds(base, nch)], idx_d.at[pl.ds(0, nch)])
        for half in range(2):
            hh = h_hbm.at[half]
            for k in range(_RPT // _ZR):
                pltpu.sync_copy(zbuf, agg_sh.at[pl.ds(r0 + k * _ZR, _ZR)])
            plsc.subcore_barrier()

            pltpu.async_copy(hh.at[idx_s.at[0]], rows.at[0], sems[0])
            pltpu.async_copy(hh.at[idx_s.at[1]], rows.at[1], sems[1])

            def step(i, carry):
                c0 = 2 * i
                for b in range(2):
                    c = c0 + b
                    pltpu.make_async_copy(hh.at[idx_s.at[c]], rows.at[b],
                                          sems[b]).wait()
                    pltpu.sync_copy(rows.at[b], agg_sh.at[idx_d.at[c]],
                                    add=True)
                    pltpu.async_copy(hh.at[idx_s.at[c + 2]], rows.at[b],
                                     sems[b])
                return carry

            lax.fori_loop(0, nch // 2 - 1, step, 0)
            for b in range(2):
                c = nch - 2 + b
                pltpu.make_async_copy(hh.at[idx_s.at[c]], rows.at[b],
                                      sems[b]).wait()
                pltpu.sync_copy(rows.at[b], agg_sh.at[idx_d.at[c]], add=True)

            plsc.subcore_barrier()
            pltpu.sync_copy(agg_sh.at[pl.ds(r0, _RPT)],
                            agg_hbm.at[half].at[cid].at[pl.ds(r0, _RPT)])
            plsc.subcore_barrier()

    lax.cond(cid == 0,
             lambda: run(_CHA, sid * _CHA),
             lambda: run(_CHB, _NS * _CHA + sid * _CHB))


@functools.cache
def _agg_call():
    chm = max(_CHA, _CHB)
    return pl.kernel(
        _agg_body,
        out_type=jax.ShapeDtypeStruct((2, _NC, _NP, _DH), jnp.float32),
        mesh=plsc.VectorSubcoreMesh(core_axis_name="c", subcore_axis_name="s"),
        compiler_params=pltpu.CompilerParams(use_tc_tiling_on_sc=False),
        scratch_types=[
            pltpu.VMEM((chm, _B), jnp.int32),
            pltpu.VMEM((chm, _B), jnp.int32),
            pltpu.VMEM((2, _B, _DH), jnp.float32),
            pltpu.VMEM((_ZR, _DH), jnp.float32),
            pltpu.VMEM_SHARED((_NP, _DH), jnp.float32),
            pltpu.SemaphoreType.DMA,
            pltpu.SemaphoreType.DMA,
        ],
    )


def _mat_body(x_ref, w_ref, dego_ref, o_ref):
    d = jnp.sum(dego_ref[...], axis=(0, 2))
    s = lax.rsqrt(jnp.maximum(d, 1.0))
    h = jnp.dot(x_ref[...], w_ref[...], preferred_element_type=jnp.float32)
    h = h * s[:, None]
    o_ref[0] = h[:, :_DH]
    o_ref[1] = h[:, _DH:]


def _mat_call(xp, w, dego):
    return pl.pallas_call(
        _mat_body,
        grid=(_NP // _TB,),
        in_specs=[
            pl.BlockSpec((_TB, _D), lambda i: (i, 0)),
            pl.BlockSpec((_D, _D), lambda i: (0, 0)),
            pl.BlockSpec((_NC, _TB, _DW), lambda i: (0, i, 0)),
        ],
        out_specs=pl.BlockSpec((2, _TB, _DH), lambda i: (0, i, 0)),
        out_shape=jax.ShapeDtypeStruct((2, _NP, _DH), jnp.float32),
    )(xp, w, dego)


def _fin_body(agg_ref, degi_ref, b_ref, o_ref):
    a = jnp.concatenate(
        [agg_ref[0, 0] + agg_ref[0, 1], agg_ref[1, 0] + agg_ref[1, 1]],
        axis=-1)
    d = jnp.sum(degi_ref[...], axis=(0, 2))
    s = lax.rsqrt(jnp.maximum(d, 1.0))
    o_ref[...] = jnp.maximum(a * s[:, None] + b_ref[...], 0.0)


def _fin_call(aggp, degi, b2):
    return pl.pallas_call(
        _fin_body,
        grid=(_NP // _TB,),
        in_specs=[
            pl.BlockSpec((2, _NC, _TB, _DH), lambda i: (0, 0, i, 0)),
            pl.BlockSpec((_NC, _TB, _DW), lambda i: (0, i, 0)),
            pl.BlockSpec((1, _D), lambda i: (0, 0)),
        ],
        out_specs=pl.BlockSpec((_TB, _D), lambda i: (i, 0)),
        out_shape=jax.ShapeDtypeStruct((_NP, _D), jnp.float32),
    )(aggp, degi, b2)


def kernel(x, edge_index, W, b):
    pad = _EP - _E
    srcp = jnp.concatenate(
        [edge_index[0], jnp.full((pad,), _N, jnp.int32)]).reshape(_NCH, _B)
    dstp = jnp.concatenate(
        [edge_index[1], jnp.full((pad,), _N, jnp.int32)]).reshape(_NCH, _B)
    xp = jnp.zeros((_NP, _D), jnp.float32).at[:_N].set(x)
    ones_blk = jnp.zeros((_B, _DW), jnp.float32).at[:, 0].set(1.0)
    zdeg = jnp.zeros((_RPT, _DW), jnp.float32)

    dego, degi = _deg_call()(srcp, dstp, ones_blk, zdeg)
    hp = _mat_call(xp, W, dego)
    aggp = _agg_call()(hp, srcp, dstp)
    outp = _fin_call(aggp, degi, b.reshape(1, _D))
    return outp[:_N]
